# baseline (speedup 1.0000x reference)
import functools

import jax
import jax.numpy as jnp
from jax import lax
from jax.experimental import pallas as pl
from jax.experimental.pallas import tpu as pltpu
from jax.experimental.pallas import tpu_sc as plsc

VOCAB = 1000000
EMBED_DIM = 32
BATCH = 4096
HIST = 200

NC = 2
NS = 16
NW = NC * NS

B = BATCH * HIST
GROUP = 128
NGROUPS = B // GROUP
GROUPS_PER_W = NGROUPS // NW
OROWS = B * EMBED_DIM // 128

_mesh = plsc.VectorSubcoreMesh(
    core_axis_name="c", subcore_axis_name="s", num_cores=NC, num_subcores=NS
)


@functools.partial(
    pl.kernel,
    out_type=jax.ShapeDtypeStruct((OROWS, 128), jnp.float32),
    mesh=_mesh,
    scratch_types=[
        pltpu.VMEM((GROUPS_PER_W, GROUP), jnp.int32),
        pltpu.VMEM((2, GROUP, EMBED_DIM), jnp.float32),
        pltpu.VMEM((2, EMBED_DIM, GROUP), jnp.float32),
        pltpu.SemaphoreType.DMA,
        pltpu.SemaphoreType.DMA,
        pltpu.SemaphoreType.DMA,
        pltpu.SemaphoreType.DMA,
    ],
    compiler_params=pltpu.CompilerParams(
        use_tc_tiling_on_sc=False, needs_layout_passes=False
    ),
)
def _emb_lookup(idx_hbm, table_hbm, out_hbm, idx_v, rows_v, blk_v, g0, g1, s0, s1):
    wid = lax.axis_index("s") * NC + lax.axis_index("c")
    g_base = wid * GROUPS_PER_W
    gsems = (g0, g1)
    ssems = (s0, s1)
    lanes = lax.iota(jnp.int32, 16)

    pltpu.sync_copy(idx_hbm.at[pl.ds(g_base, GROUPS_PER_W)], idx_v)

    def out_base(gl):
        g = g_base + gl
        return (g // 32) * 1024 + (g % 32) * 8

    def start_gather(gl, buf):
        pltpu.async_copy(table_hbm.at[idx_v.at[gl]], rows_v.at[buf], gsems[buf])

    def wait_gather(gl, buf):
        pltpu.make_async_copy(
            table_hbm.at[idx_v.at[gl]], rows_v.at[buf], gsems[buf]
        ).wait()

    def shuffle(buf):
        rows = rows_v.at[buf]
        blk = blk_v.at[buf]

        @pl.loop(0, EMBED_DIM)
        def _e(e):
            ev = jnp.broadcast_to(e, (16,)).astype(jnp.int32)
            for k in range(GROUP // 16):
                v = plsc.load_gather(rows, [lanes + k * 16, ev])
                blk[e, pl.ds(k * 16, 16)] = v

    def start_stores(gl, buf):
        ob = out_base(gl)
        for eblk in range(4):
            pltpu.async_copy(
                blk_v.at[buf, pl.ds(eblk * 8, 8)],
                out_hbm.at[pl.ds(ob + eblk * 256, 8)],
                ssems[buf],
            )

    def wait_stores(gl, buf):
        ob = out_base(gl)
        for eblk in range(4):
            pltpu.make_async_copy(
                blk_v.at[buf, pl.ds(eblk * 8, 8)],
                out_hbm.at[pl.ds(ob + eblk * 256, 8)],
                ssems[buf],
            ).wait()

    start_gather(0, 0)

    @pl.loop(0, GROUPS_PER_W // 2)
    def _pair(p):
        gl0 = p * 2
        start_gather(gl0 + 1, 1)
        wait_gather(gl0, 0)
        @pl.when(p > 0)
        def _():
            wait_stores(gl0 - 2, 0)
        shuffle(0)
        start_stores(gl0, 0)
        @pl.when(p < GROUPS_PER_W // 2 - 1)
        def _():
            start_gather(gl0 + 2, 0)
        wait_gather(gl0 + 1, 1)
        @pl.when(p > 0)
        def _():
            wait_stores(gl0 - 1, 1)
        shuffle(1)
        start_stores(gl0 + 1, 1)

    wait_stores(GROUPS_PER_W - 2, 0)
    wait_stores(GROUPS_PER_W - 1, 1)


def kernel(inputs, embedding_matrix):
    idx_t = inputs.T.reshape(NGROUPS, GROUP).astype(jnp.int32)
    out2d = _emb_lookup(idx_t, embedding_matrix)
    r5 = out2d.reshape(HIST, EMBED_DIM // 8, BATCH // 128, 8, 128)
    return r5.transpose(2, 4, 0, 1, 3).reshape(BATCH, HIST, EMBED_DIM)


# speedup vs baseline: 1.1413x; 1.1413x over previous
import functools

import jax
import jax.numpy as jnp
from jax import lax
from jax.experimental import pallas as pl
from jax.experimental.pallas import tpu as pltpu
from jax.experimental.pallas import tpu_sc as plsc

VOCAB = 1000000
EMBED_DIM = 32
BATCH = 4096
HIST = 200

NC = 2
NS = 16
NW = NC * NS

ROWS_PER_W = BATCH // NW
GROUP_SPLITS = ((0, 128), (128, 72))
ROWS_PER_CHUNK = 4
CHUNK = ROWS_PER_CHUNK * HIST
N_CHUNKS = ROWS_PER_W // ROWS_PER_CHUNK

_mesh = plsc.VectorSubcoreMesh(
    core_axis_name="c", subcore_axis_name="s", num_cores=NC, num_subcores=NS
)


@functools.partial(
    pl.kernel,
    out_type=jax.ShapeDtypeStruct((BATCH * HIST, EMBED_DIM), jnp.float32),
    mesh=_mesh,
    scratch_types=[
        pltpu.VMEM((ROWS_PER_W, HIST), jnp.int32),
        pltpu.VMEM((2, CHUNK, EMBED_DIM), jnp.float32),
        pltpu.SemaphoreType.DMA,
        pltpu.SemaphoreType.DMA,
        pltpu.SemaphoreType.DMA,
        pltpu.SemaphoreType.DMA,
    ],
    compiler_params=pltpu.CompilerParams(use_tc_tiling_on_sc=False),
)
def _emb_lookup(idx_hbm, table_hbm, out_hbm, idx_v, rows_v, g0, g1, s0, s1):
    wid = lax.axis_index("s") * NC + lax.axis_index("c")
    row0 = wid * ROWS_PER_W
    gsems = (g0, g1)
    ssems = (s0, s1)

    pltpu.sync_copy(idx_hbm.at[pl.ds(row0, ROWS_PER_W)], idx_v)

    def gathers(c, buf):
        out = []
        for j in range(ROWS_PER_CHUNK):
            for off, size in GROUP_SPLITS:
                out.append(
                    (
                        table_hbm.at[
                            idx_v.at[c * ROWS_PER_CHUNK + j, pl.ds(off, size)]
                        ],
                        rows_v.at[buf, pl.ds(j * HIST + off, size)],
                        gsems[buf],
                    )
                )
        return out

    def start_gathers(c, buf):
        for src, dst, sem in gathers(c, buf):
            pltpu.async_copy(src, dst, sem)

    def wait_gathers(c, buf):
        for src, dst, sem in gathers(c, buf):
            pltpu.make_async_copy(src, dst, sem).wait()

    def store(c, buf):
        pltpu.async_copy(
            rows_v.at[buf],
            out_hbm.at[pl.ds((row0 + c * ROWS_PER_CHUNK) * HIST, CHUNK)],
            ssems[buf],
        )

    def wait_store(c, buf):
        pltpu.make_async_copy(
            rows_v.at[buf],
            out_hbm.at[pl.ds((row0 + c * ROWS_PER_CHUNK) * HIST, CHUNK)],
            ssems[buf],
        ).wait()

    start_gathers(0, 0)

    @pl.loop(0, N_CHUNKS // 2)
    def _body(p):
        c = p * 2
        start_gathers(c + 1, 1)
        wait_gathers(c, 0)
        store(c, 0)

        @pl.when(p < N_CHUNKS // 2 - 1)
        def _():
            wait_store(c, 0)
            start_gathers(c + 2, 0)

        wait_gathers(c + 1, 1)
        store(c + 1, 1)

        @pl.when(p < N_CHUNKS // 2 - 1)
        def _():
            wait_store(c + 1, 1)

    wait_store(N_CHUNKS - 2, 0)
    wait_store(N_CHUNKS - 1, 1)


def kernel(inputs, embedding_matrix):
    out = _emb_lookup(inputs.astype(jnp.int32), embedding_matrix)
    return out.reshape(BATCH, HIST, EMBED_DIM)
